# baseline (speedup 1.0000x reference)
import jax
import jax.numpy as jnp
from jax import lax
from jax.experimental import pallas as pl
from jax.experimental.pallas import tpu as pltpu
from jax.experimental.pallas import tpu_sc as plsc

VOCAB = 1000000
DIM = 64
BATCH = 16384
NNEG = 20

NC = 2
NS = 16
LANES = 16
NW = NC * NS
B_PER_W = BATCH // NW
GROUP = 32
NGROUP = B_PER_W // GROUP
NEG_CHUNK = 128
NEG_STREAMS = GROUP * NNEG // NEG_CHUNK
TROW = 128


def _sc_body(pu_row_hbm, pu_off_hbm, pv_row_hbm, pv_off_hbm,
             ng_row_hbm, ng_off_hbm, u_tbl, v_tbl,
             pos_out, neg_out,
             pu_idx, pu_off, pv_idx, pv_off, ng_idx, ng_off,
             u_rows, v_rows, n_rows, pos_s, neg_s, sem):
  wid = lax.axis_index("s") * NC + lax.axis_index("c")
  wbase = wid * B_PER_W

  pltpu.sync_copy(pu_row_hbm.at[pl.ds(wbase, B_PER_W)], pu_idx)
  pltpu.sync_copy(pu_off_hbm.at[pl.ds(wbase, B_PER_W)], pu_off)
  pltpu.sync_copy(pv_row_hbm.at[pl.ds(wbase, B_PER_W)], pv_idx)
  pltpu.sync_copy(pv_off_hbm.at[pl.ds(wbase, B_PER_W)], pv_off)
  pltpu.sync_copy(ng_row_hbm.at[pl.ds(wbase * NNEG, B_PER_W * NNEG)], ng_idx)
  pltpu.sync_copy(ng_off_hbm.at[pl.ds(wbase * NNEG, B_PER_W * NNEG)], ng_off)

  def group_body(g, carry):
    b0 = g * GROUP
    copies = [
        pltpu.async_copy(u_tbl.at[pu_idx.at[pl.ds(b0, GROUP)]], u_rows, sem),
        pltpu.async_copy(v_tbl.at[pv_idx.at[pl.ds(b0, GROUP)]], v_rows, sem),
    ]
    for j in range(NEG_STREAMS):
      copies.append(
          pltpu.async_copy(
              v_tbl.at[ng_idx.at[pl.ds(b0 * NNEG + j * NEG_CHUNK, NEG_CHUNK)]],
              n_rows.at[pl.ds(j * NEG_CHUNK, NEG_CHUNK)], sem))
    for c in copies:
      c.wait()

    def blk_body(bb, carry2):
      uoff = pu_off[pl.ds(b0 + bb * LANES, LANES)]
      voff = pv_off[pl.ds(b0 + bb * LANES, LANES)]
      noffs = [ng_off[pl.ds((b0 + bb * LANES) * NNEG + k * LANES, LANES)]
               for k in range(NNEG * LANES // LANES)]
      for lane in range(LANES):
        b = bb * LANES + lane
        uo = uoff[lane]
        u = [u_rows[b, pl.ds(uo + j * LANES, LANES)] for j in range(4)]
        vo = voff[lane]
        p = u[0] * v_rows[b, pl.ds(vo, LANES)]
        for j in range(1, 4):
          p = p + u[j] * v_rows[b, pl.ds(vo + j * LANES, LANES)]
        nb = b * NNEG
        pos0 = lane * NNEG
        no = noffs[pos0 // LANES][pos0 % LANES]
        acc = [n_rows[nb, pl.ds(no + j * LANES, LANES)] for j in range(4)]
        for n in range(1, NNEG):
          posn = pos0 + n
          no = noffs[posn // LANES][posn % LANES]
          for j in range(4):
            acc[j] = acc[j] + n_rows[nb + n, pl.ds(no + j * LANES, LANES)]
        q = acc[0] * u[0]
        for j in range(1, 4):
          q = q + acc[j] * u[j]
        pos_s[pl.ds(b * LANES, LANES)] = p
        neg_s[pl.ds(b * LANES, LANES)] = q
      return carry2

    lax.fori_loop(0, GROUP // LANES, blk_body, 0)

    pltpu.sync_copy(pos_s, pos_out.at[pl.ds((wbase + b0) * LANES,
                                            GROUP * LANES)])
    pltpu.sync_copy(neg_s, neg_out.at[pl.ds((wbase + b0) * LANES,
                                            GROUP * LANES)])
    return carry

  lax.fori_loop(0, NGROUP, group_body, 0)


@jax.jit
def _sc_scores(pu_row, pu_off, pv_row, pv_off, ng_row, ng_off,
               u_tbl, v_tbl):
  mesh = plsc.VectorSubcoreMesh(
      core_axis_name="c", subcore_axis_name="s",
      num_cores=NC, num_subcores=NS)
  return pl.kernel(
      _sc_body,
      out_type=(
          jax.ShapeDtypeStruct((BATCH * LANES,), jnp.float32),
          jax.ShapeDtypeStruct((BATCH * LANES,), jnp.float32),
      ),
      mesh=mesh,
      scratch_types=[
          pltpu.VMEM((B_PER_W,), jnp.int32),
          pltpu.VMEM((B_PER_W,), jnp.int32),
          pltpu.VMEM((B_PER_W,), jnp.int32),
          pltpu.VMEM((B_PER_W,), jnp.int32),
          pltpu.VMEM((B_PER_W * NNEG,), jnp.int32),
          pltpu.VMEM((B_PER_W * NNEG,), jnp.int32),
          pltpu.VMEM((GROUP, TROW), jnp.float32),
          pltpu.VMEM((GROUP, TROW), jnp.float32),
          pltpu.VMEM((GROUP * NNEG, TROW), jnp.float32),
          pltpu.VMEM((GROUP * LANES,), jnp.float32),
          pltpu.VMEM((GROUP * LANES,), jnp.float32),
          pltpu.SemaphoreType.DMA,
      ],
      name="w2v_sc_gather_score",
  )(pu_row, pu_off, pv_row, pv_off, ng_row, ng_off, u_tbl, v_tbl)


FOLD_ROWS = 5000


def _tc_fold_body(lo_ref, hi_ref, o_ref):
  o_ref[:, 0:DIM] = lo_ref[...]
  o_ref[:, DIM:2 * DIM] = hi_ref[...]


@jax.jit
def _tc_fold(table):
  half_blocks = (VOCAB // 2) // FOLD_ROWS
  return pl.pallas_call(
      _tc_fold_body,
      grid=(half_blocks,),
      in_specs=[
          pl.BlockSpec((FOLD_ROWS, DIM), lambda i: (i, 0)),
          pl.BlockSpec((FOLD_ROWS, DIM), lambda i, hb=half_blocks: (i + hb, 0)),
      ],
      out_specs=pl.BlockSpec((FOLD_ROWS, 2 * DIM), lambda i: (i, 0)),
      out_shape=jax.ShapeDtypeStruct((VOCAB // 2, 2 * DIM), jnp.float32),
  )(table, table)


def _tc_loss_body(p_ref, n_ref, o_ref):
  r = lax.broadcasted_iota(jnp.int32, (128, 8), 0)
  c = lax.broadcasted_iota(jnp.int32, (128, 8), 1)
  fold = (r // LANES == c).astype(jnp.float32)
  p = jnp.dot(p_ref[...], fold)
  n = -jnp.dot(n_ref[...], fold)
  lp = jnp.minimum(p, 0.0) - jnp.log1p(jnp.exp(-jnp.abs(p)))
  ln = jnp.minimum(n, 0.0) - jnp.log1p(jnp.exp(-jnp.abs(n)))
  o_ref[0, 0] = -(jnp.sum(lp) + jnp.sum(ln))


@jax.jit
def _tc_loss(pos_s, neg_s):
  out = pl.pallas_call(
      _tc_loss_body,
      out_shape=jax.ShapeDtypeStruct((1, 1), jnp.float32),
      out_specs=pl.BlockSpec(memory_space=pltpu.SMEM),
  )(pos_s.reshape(2048, 128), neg_s.reshape(2048, 128))
  return out[0, 0]


def kernel(pos_u, pos_v, neg_v, u_table, v_table):
  u_tbl = _tc_fold(u_table)
  v_tbl = _tc_fold(v_table)
  neg_flat = neg_v.reshape(-1)
  half = VOCAB // 2
  pos_s, neg_s = _sc_scores(
      pos_u % half, (pos_u // half) * DIM,
      pos_v % half, (pos_v // half) * DIM,
      neg_flat % half, (neg_flat // half) * DIM,
      u_tbl, v_tbl)
  return _tc_loss(pos_s, neg_s)


# speedup vs baseline: 1.1913x; 1.1913x over previous
import jax
import jax.numpy as jnp
from jax import lax
from jax.experimental import pallas as pl
from jax.experimental.pallas import tpu as pltpu
from jax.experimental.pallas import tpu_sc as plsc

VOCAB = 1000000
DIM = 64
BATCH = 16384
NNEG = 20

NC = 2
NS = 16
LANES = 16
NW = NC * NS
B_PER_W = BATCH // NW
GROUP = 64
NGROUP = B_PER_W // GROUP
NEG_CHUNK = 128
NEG_STREAMS = GROUP * NNEG // NEG_CHUNK
UROW = 128
HALF = VOCAB // 2


def _sc_body(pu_row_hbm, pu_off_hbm, pos_v_hbm, neg_flat_hbm,
             u_tbl, v_table,
             pos_out, neg_out,
             pu_idx, pu_off, pv_idx, ng_idx,
             u_rows, v_rows, n_rows, pos_s, neg_s, sem):
  wid = lax.axis_index("s") * NC + lax.axis_index("c")
  wbase = wid * B_PER_W

  pltpu.sync_copy(pu_row_hbm.at[pl.ds(wbase, B_PER_W)], pu_idx)
  pltpu.sync_copy(pu_off_hbm.at[pl.ds(wbase, B_PER_W)], pu_off)
  pltpu.sync_copy(pos_v_hbm.at[pl.ds(wbase, B_PER_W)], pv_idx)
  pltpu.sync_copy(neg_flat_hbm.at[pl.ds(wbase * NNEG, B_PER_W * NNEG)],
                  ng_idx)

  def group_body(g, carry):
    b0 = g * GROUP
    copies = [
        pltpu.async_copy(u_tbl.at[pu_idx.at[pl.ds(b0, GROUP)]], u_rows, sem),
        pltpu.async_copy(v_table.at[pv_idx.at[pl.ds(b0, GROUP)]], v_rows,
                         sem),
    ]
    for j in range(NEG_STREAMS):
      copies.append(
          pltpu.async_copy(
              v_table.at[ng_idx.at[pl.ds(b0 * NNEG + j * NEG_CHUNK,
                                         NEG_CHUNK)]],
              n_rows.at[pl.ds(j * NEG_CHUNK, NEG_CHUNK)], sem))
    for c in copies:
      c.wait()

    def blk_body(bb, carry2):
      uoff = pu_off[pl.ds(b0 + bb * LANES, LANES)]
      for lane in range(LANES):
        b = bb * LANES + lane
        uo = uoff[lane]
        u = [u_rows[b, pl.ds(uo + j * LANES, LANES)] for j in range(4)]
        p = u[0] * v_rows[b, pl.ds(0, LANES)]
        for j in range(1, 4):
          p = p + u[j] * v_rows[b, pl.ds(j * LANES, LANES)]
        nb = b * NNEG
        acc = [n_rows[nb, pl.ds(j * LANES, LANES)] for j in range(4)]
        for n in range(1, NNEG):
          for j in range(4):
            acc[j] = acc[j] + n_rows[nb + n, pl.ds(j * LANES, LANES)]
        q = acc[0] * u[0]
        for j in range(1, 4):
          q = q + acc[j] * u[j]
        pos_s[pl.ds(b * LANES, LANES)] = p
        neg_s[pl.ds(b * LANES, LANES)] = q
      return carry2

    lax.fori_loop(0, GROUP // LANES, blk_body, 0)

    pltpu.sync_copy(pos_s, pos_out.at[pl.ds((wbase + b0) * LANES,
                                            GROUP * LANES)])
    pltpu.sync_copy(neg_s, neg_out.at[pl.ds((wbase + b0) * LANES,
                                            GROUP * LANES)])
    return carry

  lax.fori_loop(0, NGROUP, group_body, 0)


@jax.jit
def _sc_scores(pu_row, pu_off, pos_v, neg_flat, u_tbl, v_table):
  mesh = plsc.VectorSubcoreMesh(
      core_axis_name="c", subcore_axis_name="s",
      num_cores=NC, num_subcores=NS)
  return pl.kernel(
      _sc_body,
      out_type=(
          jax.ShapeDtypeStruct((BATCH * LANES,), jnp.float32),
          jax.ShapeDtypeStruct((BATCH * LANES,), jnp.float32),
      ),
      mesh=mesh,
      scratch_types=[
          pltpu.VMEM((B_PER_W,), jnp.int32),
          pltpu.VMEM((B_PER_W,), jnp.int32),
          pltpu.VMEM((B_PER_W,), jnp.int32),
          pltpu.VMEM((B_PER_W * NNEG,), jnp.int32),
          pltpu.VMEM((GROUP, UROW), jnp.float32),
          pltpu.VMEM((GROUP, DIM), jnp.float32),
          pltpu.VMEM((GROUP * NNEG, DIM), jnp.float32),
          pltpu.VMEM((GROUP * LANES,), jnp.float32),
          pltpu.VMEM((GROUP * LANES,), jnp.float32),
          pltpu.SemaphoreType.DMA,
      ],
      compiler_params=pltpu.CompilerParams(use_tc_tiling_on_sc=False),
      name="w2v_sc_gather_score",
  )(pu_row, pu_off, pos_v, neg_flat, u_tbl, v_table)


FOLD_ROWS = 5000


def _tc_fold_body(lo_ref, hi_ref, o_ref):
  o_ref[:, 0:DIM] = lo_ref[...]
  o_ref[:, DIM:2 * DIM] = hi_ref[...]


@jax.jit
def _tc_fold(table):
  half_blocks = HALF // FOLD_ROWS
  return pl.pallas_call(
      _tc_fold_body,
      grid=(half_blocks,),
      in_specs=[
          pl.BlockSpec((FOLD_ROWS, DIM), lambda i: (i, 0)),
          pl.BlockSpec((FOLD_ROWS, DIM), lambda i, hb=half_blocks: (i + hb, 0)),
      ],
      out_specs=pl.BlockSpec((FOLD_ROWS, 2 * DIM), lambda i: (i, 0)),
      out_shape=jax.ShapeDtypeStruct((HALF, 2 * DIM), jnp.float32),
  )(table, table)


def _tc_loss_body(p_ref, n_ref, o_ref):
  r = lax.broadcasted_iota(jnp.int32, (128, 8), 0)
  c = lax.broadcasted_iota(jnp.int32, (128, 8), 1)
  fold = (r // LANES == c).astype(jnp.float32)
  p = jnp.dot(p_ref[...], fold)
  n = -jnp.dot(n_ref[...], fold)
  lp = jnp.minimum(p, 0.0) - jnp.log1p(jnp.exp(-jnp.abs(p)))
  ln = jnp.minimum(n, 0.0) - jnp.log1p(jnp.exp(-jnp.abs(n)))
  o_ref[0, 0] = -(jnp.sum(lp) + jnp.sum(ln))


@jax.jit
def _tc_loss(pos_s, neg_s):
  out = pl.pallas_call(
      _tc_loss_body,
      out_shape=jax.ShapeDtypeStruct((1, 1), jnp.float32),
      out_specs=pl.BlockSpec(memory_space=pltpu.SMEM),
  )(pos_s.reshape(2048, 128), neg_s.reshape(2048, 128))
  return out[0, 0]


def kernel(pos_u, pos_v, neg_v, u_table, v_table):
  u_tbl = _tc_fold(u_table)
  neg_flat = neg_v.reshape(-1)
  pos_s, neg_s = _sc_scores(
      pos_u % HALF, (pos_u // HALF) * DIM,
      pos_v, neg_flat, u_tbl, v_table)
  return _tc_loss(pos_s, neg_s)


# speedup vs baseline: 1.1913x; 1.0000x over previous
import jax
import jax.numpy as jnp
from jax import lax
from jax.experimental import pallas as pl
from jax.experimental.pallas import tpu as pltpu
from jax.experimental.pallas import tpu_sc as plsc

VOCAB = 1000000
DIM = 64
BATCH = 16384
NNEG = 20

NC = 2
NS = 16
LANES = 16
NW = NC * NS
B_PER_W = BATCH // NW
GROUP = 64
NGROUP = B_PER_W // GROUP
NEG_CHUNK = 128
NEG_STREAMS = GROUP * NNEG // NEG_CHUNK
UROW = 128
HALF = VOCAB // 2


def _sc_body(pu_row_hbm, pu_off_hbm, pos_v_hbm, neg_flat_hbm,
             u_tbl, v_table,
             pos_out, neg_out,
             pu_idx, pu_off, pv_idx, ng_idx,
             u_rows, v_rows, n_rows, pos_s, neg_s, sem):
  wid = lax.axis_index("s") * NC + lax.axis_index("c")
  wbase = wid * B_PER_W

  pltpu.sync_copy(pu_row_hbm.at[pl.ds(wbase, B_PER_W)], pu_idx)
  pltpu.sync_copy(pu_off_hbm.at[pl.ds(wbase, B_PER_W)], pu_off)
  pltpu.sync_copy(pos_v_hbm.at[pl.ds(wbase, B_PER_W)], pv_idx)
  pltpu.sync_copy(neg_flat_hbm.at[pl.ds(wbase * NNEG, B_PER_W * NNEG)],
                  ng_idx)

  def group_body(g, carry):
    b0 = g * GROUP
    copies = [
        pltpu.async_copy(u_tbl.at[pu_idx.at[pl.ds(b0, GROUP)]], u_rows, sem),
        pltpu.async_copy(v_table.at[pv_idx.at[pl.ds(b0, GROUP)]], v_rows,
                         sem),
    ]
    for j in range(NEG_STREAMS):
      copies.append(
          pltpu.async_copy(
              v_table.at[ng_idx.at[pl.ds(b0 * NNEG + j * NEG_CHUNK,
                                         NEG_CHUNK)]],
              n_rows.at[pl.ds(j * NEG_CHUNK, NEG_CHUNK)], sem))
    for c in copies:
      c.wait()

    def blk_body(bb, carry2):
      uoff = pu_off[pl.ds(b0 + bb * LANES, LANES)]
      for lane in range(LANES):
        b = bb * LANES + lane
        uo = uoff[lane]
        u = [u_rows[b, pl.ds(uo + j * LANES, LANES)] for j in range(4)]
        p = u[0] * v_rows[b, pl.ds(0, LANES)]
        for j in range(1, 4):
          p = p + u[j] * v_rows[b, pl.ds(j * LANES, LANES)]
        nb = b * NNEG
        acc = [n_rows[nb, pl.ds(j * LANES, LANES)] for j in range(4)]
        for n in range(1, NNEG):
          for j in range(4):
            acc[j] = acc[j] + n_rows[nb + n, pl.ds(j * LANES, LANES)]
        q = acc[0] * u[0]
        for j in range(1, 4):
          q = q + acc[j] * u[j]
        pos_s[pl.ds(b * LANES, LANES)] = p
        neg_s[pl.ds(b * LANES, LANES)] = q
      return carry2

    lax.fori_loop(0, GROUP // LANES, blk_body, 0)

    pltpu.sync_copy(pos_s, pos_out.at[pl.ds((wbase + b0) * LANES,
                                            GROUP * LANES)])
    pltpu.sync_copy(neg_s, neg_out.at[pl.ds((wbase + b0) * LANES,
                                            GROUP * LANES)])
    return carry

  lax.fori_loop(0, NGROUP, group_body, 0)


def _sc_scores(pu_row, pu_off, pos_v, neg_flat, u_tbl, v_table):
  mesh = plsc.VectorSubcoreMesh(
      core_axis_name="c", subcore_axis_name="s",
      num_cores=NC, num_subcores=NS)
  return pl.kernel(
      _sc_body,
      out_type=(
          jax.ShapeDtypeStruct((BATCH * LANES,), jnp.float32),
          jax.ShapeDtypeStruct((BATCH * LANES,), jnp.float32),
      ),
      mesh=mesh,
      scratch_types=[
          pltpu.VMEM((B_PER_W,), jnp.int32),
          pltpu.VMEM((B_PER_W,), jnp.int32),
          pltpu.VMEM((B_PER_W,), jnp.int32),
          pltpu.VMEM((B_PER_W * NNEG,), jnp.int32),
          pltpu.VMEM((GROUP, UROW), jnp.float32),
          pltpu.VMEM((GROUP, DIM), jnp.float32),
          pltpu.VMEM((GROUP * NNEG, DIM), jnp.float32),
          pltpu.VMEM((GROUP * LANES,), jnp.float32),
          pltpu.VMEM((GROUP * LANES,), jnp.float32),
          pltpu.SemaphoreType.DMA,
      ],
      compiler_params=pltpu.CompilerParams(use_tc_tiling_on_sc=False),
      name="w2v_sc_gather_score",
  )(pu_row, pu_off, pos_v, neg_flat, u_tbl, v_table)


FOLD_ROWS = 5000


def _tc_fold_body(lo_ref, hi_ref, o_ref):
  o_ref[:, 0:DIM] = lo_ref[...]
  o_ref[:, DIM:2 * DIM] = hi_ref[...]


def _tc_fold(table):
  half_blocks = HALF // FOLD_ROWS
  return pl.pallas_call(
      _tc_fold_body,
      grid=(half_blocks,),
      in_specs=[
          pl.BlockSpec((FOLD_ROWS, DIM), lambda i: (i, 0)),
          pl.BlockSpec((FOLD_ROWS, DIM), lambda i, hb=half_blocks: (i + hb, 0)),
      ],
      out_specs=pl.BlockSpec((FOLD_ROWS, 2 * DIM), lambda i: (i, 0)),
      out_shape=jax.ShapeDtypeStruct((HALF, 2 * DIM), jnp.float32),
  )(table, table)


def _tc_loss_body(p_ref, n_ref, o_ref):
  r = lax.broadcasted_iota(jnp.int32, (128, 8), 0)
  c = lax.broadcasted_iota(jnp.int32, (128, 8), 1)
  fold = (r // LANES == c).astype(jnp.float32)
  p = jnp.dot(p_ref[...], fold)
  n = -jnp.dot(n_ref[...], fold)
  lp = jnp.minimum(p, 0.0) - jnp.log1p(jnp.exp(-jnp.abs(p)))
  ln = jnp.minimum(n, 0.0) - jnp.log1p(jnp.exp(-jnp.abs(n)))
  o_ref[0, 0] = -(jnp.sum(lp) + jnp.sum(ln))


def _tc_loss(pos_s, neg_s):
  out = pl.pallas_call(
      _tc_loss_body,
      out_shape=jax.ShapeDtypeStruct((1, 1), jnp.float32),
      out_specs=pl.BlockSpec(memory_space=pltpu.SMEM),
  )(pos_s.reshape(2048, 128), neg_s.reshape(2048, 128))
  return out[0, 0]


@jax.jit
def _impl(pos_u, pos_v, neg_v, u_table, v_table):
  u_tbl = _tc_fold(u_table)
  neg_flat = neg_v.reshape(-1)
  pos_s, neg_s = _sc_scores(
      pos_u % HALF, (pos_u // HALF) * DIM,
      pos_v, neg_flat, u_tbl, v_table)
  return _tc_loss(pos_s, neg_s)


def kernel(pos_u, pos_v, neg_v, u_table, v_table):
  return _impl(pos_u, pos_v, neg_v, u_table, v_table)


# speedup vs baseline: 1.2571x; 1.0552x over previous
import functools

import jax
import jax.numpy as jnp
from jax import lax
from jax.experimental import pallas as pl
from jax.experimental.pallas import tpu as pltpu
from jax.experimental.pallas import tpu_sc as plsc

VOCAB = 1000000
DIM = 64
BATCH = 16384
NNEG = 20

NC = 2
NS = 16
LANES = 16
NW = NC * NS
B_PER_W = BATCH // NW
GROUP = 32
NGROUP = B_PER_W // GROUP
NEG_CHUNK = 128
NEG_STREAMS = GROUP * NNEG // NEG_CHUNK


def _sc_body(pos_u_hbm, pos_v_hbm, neg_flat_hbm, u_table, v_table,
             pos_out, neg_out,
             pu_idx, pv_idx, ng_idx,
             u_rows0, v_rows0, n_rows0, u_rows1, v_rows1, n_rows1,
             pos_s, neg_s, sem0, sem1):
  wid = lax.axis_index("s") * NC + lax.axis_index("c")
  wbase = wid * B_PER_W

  pltpu.sync_copy(pos_u_hbm.at[pl.ds(wbase, B_PER_W)], pu_idx)
  pltpu.sync_copy(pos_v_hbm.at[pl.ds(wbase, B_PER_W)], pv_idx)
  pltpu.sync_copy(neg_flat_hbm.at[pl.ds(wbase * NNEG, B_PER_W * NNEG)],
                  ng_idx)

  bufs = ((u_rows0, v_rows0, n_rows0, sem0),
          (u_rows1, v_rows1, n_rows1, sem1))

  def fire(g, bufset):
    u_rows, v_rows, n_rows, sem = bufset
    b0 = g * GROUP
    copies = [
        pltpu.async_copy(u_table.at[pu_idx.at[pl.ds(b0, GROUP)]], u_rows,
                         sem),
        pltpu.async_copy(v_table.at[pv_idx.at[pl.ds(b0, GROUP)]], v_rows,
                         sem),
    ]
    for j in range(NEG_STREAMS):
      copies.append(
          pltpu.async_copy(
              v_table.at[ng_idx.at[pl.ds(b0 * NNEG + j * NEG_CHUNK,
                                         NEG_CHUNK)]],
              n_rows.at[pl.ds(j * NEG_CHUNK, NEG_CHUNK)], sem))
    return copies

  def compute(g, bufset):
    u_rows, v_rows, n_rows, _ = bufset

    def elem_body(b, carry2):
      u = [u_rows[b, pl.ds(j * LANES, LANES)] for j in range(4)]
      p = u[0] * v_rows[b, pl.ds(0, LANES)]
      for j in range(1, 4):
        p = p + u[j] * v_rows[b, pl.ds(j * LANES, LANES)]
      nb = b * NNEG
      acc = [n_rows[nb, pl.ds(j * LANES, LANES)] for j in range(4)]
      for n in range(1, NNEG):
        for j in range(4):
          acc[j] = acc[j] + n_rows[nb + n, pl.ds(j * LANES, LANES)]
      q = acc[0] * u[0]
      for j in range(1, 4):
        q = q + acc[j] * u[j]
      pos_s[b, :] = p
      neg_s[b, :] = q
      return carry2

    lax.fori_loop(0, GROUP, elem_body, 0)
    base = wbase + g * GROUP
    pltpu.sync_copy(pos_s, pos_out.at[pl.ds(base, GROUP)])
    pltpu.sync_copy(neg_s, neg_out.at[pl.ds(base, GROUP)])

  pending = fire(0, bufs[0])
  for g in range(NGROUP):
    nxt = fire(g + 1, bufs[(g + 1) % 2]) if g + 1 < NGROUP else None
    for c in pending:
      c.wait()
    compute(g, bufs[g % 2])
    pending = nxt


@jax.jit
def _sc_scores(pos_u, pos_v, neg_flat, u_table, v_table):
  mesh = plsc.VectorSubcoreMesh(
      core_axis_name="c", subcore_axis_name="s",
      num_cores=NC, num_subcores=NS)
  return pl.kernel(
      _sc_body,
      out_type=(
          jax.ShapeDtypeStruct((BATCH, LANES), jnp.float32),
          jax.ShapeDtypeStruct((BATCH, LANES), jnp.float32),
      ),
      mesh=mesh,
      scratch_types=[
          pltpu.VMEM((B_PER_W,), jnp.int32),
          pltpu.VMEM((B_PER_W,), jnp.int32),
          pltpu.VMEM((B_PER_W * NNEG,), jnp.int32),
          pltpu.VMEM((GROUP, DIM), jnp.float32),
          pltpu.VMEM((GROUP, DIM), jnp.float32),
          pltpu.VMEM((GROUP * NNEG, DIM), jnp.float32),
          pltpu.VMEM((GROUP, DIM), jnp.float32),
          pltpu.VMEM((GROUP, DIM), jnp.float32),
          pltpu.VMEM((GROUP * NNEG, DIM), jnp.float32),
          pltpu.VMEM((GROUP, LANES), jnp.float32),
          pltpu.VMEM((GROUP, LANES), jnp.float32),
          pltpu.SemaphoreType.DMA,
          pltpu.SemaphoreType.DMA,
      ],
      compiler_params=pltpu.CompilerParams(use_tc_tiling_on_sc=False),
      name="w2v_sc_gather_score",
  )(pos_u, pos_v, neg_flat, u_table, v_table)


TC_ROWS = 2048


def _tc_loss_body(p_ref, n_ref, o_ref):
  i = pl.program_id(0)

  @pl.when(i == 0)
  def _():
    o_ref[0, 0] = 0.0

  p = jnp.sum(p_ref[...], axis=1)
  n = -jnp.sum(n_ref[...], axis=1)
  lp = jnp.minimum(p, 0.0) - jnp.log1p(jnp.exp(-jnp.abs(p)))
  ln = jnp.minimum(n, 0.0) - jnp.log1p(jnp.exp(-jnp.abs(n)))
  o_ref[0, 0] += -(jnp.sum(lp) + jnp.sum(ln))


@jax.jit
def _tc_loss(pos_s, neg_s):
  out = pl.pallas_call(
      _tc_loss_body,
      grid=(BATCH // TC_ROWS,),
      in_specs=[
          pl.BlockSpec((TC_ROWS, LANES), lambda i: (i, 0)),
          pl.BlockSpec((TC_ROWS, LANES), lambda i: (i, 0)),
      ],
      out_shape=jax.ShapeDtypeStruct((1, 1), jnp.float32),
      out_specs=pl.BlockSpec(memory_space=pltpu.SMEM),
  )(pos_s, neg_s)
  return out[0, 0]


def kernel(pos_u, pos_v, neg_v, u_table, v_table):
  neg_flat = neg_v.reshape(-1)
  pos_s, neg_s = _sc_scores(pos_u, pos_v, neg_flat, u_table, v_table)
  return _tc_loss(pos_s, neg_s)


# speedup vs baseline: 1.2576x; 1.0004x over previous
import functools

import jax
import jax.numpy as jnp
from jax import lax
from jax.experimental import pallas as pl
from jax.experimental.pallas import tpu as pltpu
from jax.experimental.pallas import tpu_sc as plsc

VOCAB = 1000000
DIM = 64
BATCH = 16384
NNEG = 20

NC = 2
NS = 16
LANES = 16
NW = NC * NS
B_PER_W = BATCH // NW
GROUP = 32
NGROUP = B_PER_W // GROUP
NEG_CHUNK = 128
NEG_STREAMS = GROUP * NNEG // NEG_CHUNK


def _sc_body(pos_u_hbm, pos_v_hbm, neg_flat_hbm, u_table, v_table,
             pos_out, neg_out,
             pu_idx, pv_idx, ng_idx,
             u_rows0, v_rows0, n_rows0, u_rows1, v_rows1, n_rows1,
             pos_s0, neg_s0, pos_s1, neg_s1, sem0, sem1, wsem):
  wid = lax.axis_index("s") * NC + lax.axis_index("c")
  wbase = wid * B_PER_W

  pltpu.sync_copy(pos_u_hbm.at[pl.ds(wbase, B_PER_W)], pu_idx)
  pltpu.sync_copy(pos_v_hbm.at[pl.ds(wbase, B_PER_W)], pv_idx)
  pltpu.sync_copy(neg_flat_hbm.at[pl.ds(wbase * NNEG, B_PER_W * NNEG)],
                  ng_idx)

  bufs = ((u_rows0, v_rows0, n_rows0, pos_s0, neg_s0, sem0),
          (u_rows1, v_rows1, n_rows1, pos_s1, neg_s1, sem1))

  def fire(g, bufset):
    u_rows, v_rows, n_rows, _, _, sem = bufset
    b0 = g * GROUP
    copies = [
        pltpu.async_copy(u_table.at[pu_idx.at[pl.ds(b0, GROUP)]], u_rows,
                         sem),
        pltpu.async_copy(v_table.at[pv_idx.at[pl.ds(b0, GROUP)]], v_rows,
                         sem),
    ]
    for j in range(NEG_STREAMS):
      copies.append(
          pltpu.async_copy(
              v_table.at[ng_idx.at[pl.ds(b0 * NNEG + j * NEG_CHUNK,
                                         NEG_CHUNK)]],
              n_rows.at[pl.ds(j * NEG_CHUNK, NEG_CHUNK)], sem))
    return copies

  def compute(g, bufset):
    u_rows, v_rows, n_rows, pos_s, neg_s, _ = bufset

    def elem_body(b, carry2):
      u = [u_rows[b, pl.ds(j * LANES, LANES)] for j in range(4)]
      p = u[0] * v_rows[b, pl.ds(0, LANES)]
      for j in range(1, 4):
        p = p + u[j] * v_rows[b, pl.ds(j * LANES, LANES)]
      nb = b * NNEG
      acc = [n_rows[nb, pl.ds(j * LANES, LANES)] for j in range(4)]
      for n in range(1, NNEG):
        for j in range(4):
          acc[j] = acc[j] + n_rows[nb + n, pl.ds(j * LANES, LANES)]
      q = acc[0] * u[0]
      for j in range(1, 4):
        q = q + acc[j] * u[j]
      pos_s[b, :] = p
      neg_s[b, :] = q
      return carry2

    lax.fori_loop(0, GROUP, elem_body, 0)
    base = wbase + g * GROUP
    return [
        pltpu.async_copy(pos_s, pos_out.at[pl.ds(base, GROUP)], wsem),
        pltpu.async_copy(neg_s, neg_out.at[pl.ds(base, GROUP)], wsem),
    ]

  pending = fire(0, bufs[0])
  writes = []
  for g in range(NGROUP):
    nxt = fire(g + 1, bufs[(g + 1) % 2]) if g + 1 < NGROUP else None
    for c in pending:
      c.wait()
    for w in writes:
      w.wait()
    writes = compute(g, bufs[g % 2])
    pending = nxt
  for w in writes:
    w.wait()


@jax.jit
def _sc_scores(pos_u, pos_v, neg_flat, u_table, v_table):
  mesh = plsc.VectorSubcoreMesh(
      core_axis_name="c", subcore_axis_name="s",
      num_cores=NC, num_subcores=NS)
  return pl.kernel(
      _sc_body,
      out_type=(
          jax.ShapeDtypeStruct((BATCH, LANES), jnp.float32),
          jax.ShapeDtypeStruct((BATCH, LANES), jnp.float32),
      ),
      mesh=mesh,
      scratch_types=[
          pltpu.VMEM((B_PER_W,), jnp.int32),
          pltpu.VMEM((B_PER_W,), jnp.int32),
          pltpu.VMEM((B_PER_W * NNEG,), jnp.int32),
          pltpu.VMEM((GROUP, DIM), jnp.float32),
          pltpu.VMEM((GROUP, DIM), jnp.float32),
          pltpu.VMEM((GROUP * NNEG, DIM), jnp.float32),
          pltpu.VMEM((GROUP, DIM), jnp.float32),
          pltpu.VMEM((GROUP, DIM), jnp.float32),
          pltpu.VMEM((GROUP * NNEG, DIM), jnp.float32),
          pltpu.VMEM((GROUP, LANES), jnp.float32),
          pltpu.VMEM((GROUP, LANES), jnp.float32),
          pltpu.VMEM((GROUP, LANES), jnp.float32),
          pltpu.VMEM((GROUP, LANES), jnp.float32),
          pltpu.SemaphoreType.DMA,
          pltpu.SemaphoreType.DMA,
          pltpu.SemaphoreType.DMA,
      ],
      compiler_params=pltpu.CompilerParams(use_tc_tiling_on_sc=False),
      name="w2v_sc_gather_score",
  )(pos_u, pos_v, neg_flat, u_table, v_table)


TC_ROWS = 2048


def _tc_loss_body(p_ref, n_ref, o_ref):
  i = pl.program_id(0)

  @pl.when(i == 0)
  def _():
    o_ref[0, 0] = 0.0

  p = jnp.sum(p_ref[...], axis=1)
  n = -jnp.sum(n_ref[...], axis=1)
  lp = jnp.minimum(p, 0.0) - jnp.log1p(jnp.exp(-jnp.abs(p)))
  ln = jnp.minimum(n, 0.0) - jnp.log1p(jnp.exp(-jnp.abs(n)))
  o_ref[0, 0] += -(jnp.sum(lp) + jnp.sum(ln))


@jax.jit
def _tc_loss(pos_s, neg_s):
  out = pl.pallas_call(
      _tc_loss_body,
      grid=(BATCH // TC_ROWS,),
      in_specs=[
          pl.BlockSpec((TC_ROWS, LANES), lambda i: (i, 0)),
          pl.BlockSpec((TC_ROWS, LANES), lambda i: (i, 0)),
      ],
      out_shape=jax.ShapeDtypeStruct((1, 1), jnp.float32),
      out_specs=pl.BlockSpec(memory_space=pltpu.SMEM),
  )(pos_s, neg_s)
  return out[0, 0]


def kernel(pos_u, pos_v, neg_v, u_table, v_table):
  neg_flat = neg_v.reshape(-1)
  pos_s, neg_s = _sc_scores(pos_u, pos_v, neg_flat, u_table, v_table)
  return _tc_loss(pos_s, neg_s)
